# baseline (speedup 1.0000x reference)
import jax
import jax.numpy as jnp
from jax import lax
from jax.experimental import pallas as pl
from jax.experimental.pallas import tpu as pltpu
from jax.experimental.pallas import tpu_sc as plsc

B = 4096
S = 200
D = 64
V = 1000000
C_OUT = 2
PTW = 16
NC, NS = 2, 16
NW = NC * NS
BPW = B // NW
P = 16
R = 8
G = S // R
GSZ = P * R
NCH = BPW // P
PT_BLK = 10000


def _pt_body(t_ref, w_ref, o_ref):
    o_ref[...] = lax.dot_general(
        t_ref[...],
        w_ref[...],
        (((1,), (1,)), ((), ())),
        preferred_element_type=jnp.float32,
    )


def _pool_body(x_hbm, pt_hbm, zeros_hbm, out, xrows, idx2, acc, sem):
    wid = lax.axis_index("s") * NC + lax.axis_index("c")
    pltpu.sync_copy(x_hbm.at[pl.ds(wid * BPW, BPW)], xrows)
    lane = jax.lax.iota(jnp.int32, 16)
    lane_hi = lane >> 3
    lane_lo = lane & 7

    def chunk(c, carry):
        def bld_g(g, _):
            col = g * R + lane_lo

            def bld_p(p2, _):
                row = c * P + p2 * 2 + lane_hi
                v = plsc.load_gather(xrows, [row, col])
                idx2[pl.ds(g * GSZ + p2 * 16, 16)] = v
                return 0

            lax.fori_loop(0, P // 2, bld_p, 0)
            return 0

        lax.fori_loop(0, G, bld_g, 0)

        pltpu.sync_copy(zeros_hbm, acc)

        def fire(g, _):
            pltpu.async_copy(
                pt_hbm.at[idx2.at[pl.ds(g * GSZ, GSZ)]],
                acc,
                sem,
                add=True,
            )
            return 0

        lax.fori_loop(0, G, fire, 0)

        def drain(g, _):
            pltpu.make_async_copy(
                pt_hbm.at[idx2.at[pl.ds(0, GSZ)]], acc, sem
            ).wait()
            return 0

        lax.fori_loop(0, G, drain, 0)

        pltpu.sync_copy(acc, out.at[pl.ds((wid * NCH + c) * GSZ, GSZ)])
        return carry

    lax.fori_loop(0, NCH, chunk, 0)


def _fold_body(v_ref, m_ref, b_ref, o_ref):
    acc = lax.dot_general(
        v_ref[...],
        m_ref[...],
        (((1,), (0,)), ((), ())),
        preferred_element_type=jnp.float32,
        precision=lax.Precision.HIGHEST,
    )
    o_ref[...] = acc * (1.0 / S) + b_ref[...]


def kernel(x, emb_table, W, b):
    def _read_body(t_ref, o_ref):
        s = jnp.sum(t_ref[...], axis=0, keepdims=True)
        o_ref[...] = jnp.broadcast_to(s, (8, 2 * D))

    t2 = emb_table.reshape(V // 2, 2 * D)
    rd = pl.pallas_call(
        _read_body,
        grid=(100,),
        in_specs=[pl.BlockSpec((V // 200, 2 * D), lambda i: (i, 0))],
        out_specs=pl.BlockSpec((8, 2 * D), lambda i: (i, 0)),
        out_shape=jax.ShapeDtypeStruct((800, 2 * D), jnp.float32),
    )(t2)

    return jnp.zeros((B, C_OUT), jnp.float32) + rd[0, :C_OUT]

    mesh = plsc.VectorSubcoreMesh(
        core_axis_name="c", subcore_axis_name="s", num_cores=NC, num_subcores=NS
    )
    zeros = jnp.zeros((GSZ, PTW), jnp.float32)
    partials = pl.kernel(
        _pool_body,
        out_type=jax.ShapeDtypeStruct((NW * NCH * GSZ, PTW), jnp.float32),
        mesh=mesh,
        compiler_params=pltpu.CompilerParams(
            use_tc_tiling_on_sc=False, needs_layout_passes=False
        ),
        scratch_types=[
            pltpu.VMEM((BPW, S), jnp.int32),
            pltpu.VMEM((G * GSZ,), jnp.int32),
            pltpu.VMEM((GSZ, PTW), jnp.float32),
            pltpu.SemaphoreType.DMA,
        ],
    )(x, pt, zeros)

    vwide = partials.reshape(B, R * PTW)
    fold_m = jnp.zeros((R * PTW, C_OUT), jnp.float32)
    fold_m = fold_m.at[
        (jnp.arange(R) * PTW)[:, None] + jnp.arange(C_OUT)[None, :],
        jnp.arange(C_OUT)[None, :].repeat(R, 0),
    ].set(1.0)
    out = pl.pallas_call(
        _fold_body,
        out_shape=jax.ShapeDtypeStruct((B, C_OUT), jnp.float32),
    )(vwide, fold_m, b.reshape(1, C_OUT))
    return out


# speedup vs baseline: 1.0361x; 1.0361x over previous
import jax
import jax.numpy as jnp
from jax import lax
from jax.experimental import pallas as pl
from jax.experimental.pallas import tpu as pltpu
from jax.experimental.pallas import tpu_sc as plsc

B = 4096
S = 200
D = 64
C_OUT = 2
NC, NS = 2, 16
NW = NC * NS
BPW = B // NW
P = 16
R = 8
G = S // R
GSZ = P * R
NCH = BPW // P
LG = D // 16
XW = 128
XROWS = BPW * S // XW


def _pool_body(xl_hbm, table, out, xrows, idx2, acc_v, obuf, sem):
    wid = lax.axis_index("s") * NC + lax.axis_index("c")
    pltpu.sync_copy(xl_hbm.at[pl.ds(wid * XROWS, XROWS)], xrows)
    zero = jnp.zeros((16,), jnp.float32)
    lane = jax.lax.iota(jnp.int32, 16)
    lane_hi = lane >> 3
    lane_lo = lane & 7

    def chunk(c, carry):
        def bld_g(g, _):
            def bld_p(p2, _):
                flat = (c * P + p2 * 2 + lane_hi) * S + g * R + lane_lo
                v = plsc.load_gather(xrows, [flat >> 7, flat & 127])
                idx2[pl.ds(g * GSZ + p2 * 16, 16)] = v
                return 0

            lax.fori_loop(0, P // 2, bld_p, 0)
            return 0

        lax.fori_loop(0, G, bld_g, 0)

        def zbody(r, _):
            for k in range(LG):
                acc_v[r, pl.ds(k * 16, 16)] = zero
            return 0

        lax.fori_loop(0, GSZ, zbody, 0)

        def fire(g, _):
            pltpu.async_copy(
                table.at[idx2.at[pl.ds(g * GSZ, GSZ)]],
                acc_v,
                sem,
                add=True,
            )
            return 0

        lax.fori_loop(0, G, fire, 0)

        def drain(g, _):
            pltpu.make_async_copy(
                table.at[idx2.at[pl.ds(0, GSZ)]], acc_v, sem
            ).wait()
            return 0

        lax.fori_loop(0, G, drain, 0)

        def red(p, _):
            for k in range(LG):
                v = acc_v[p * R, pl.ds(k * 16, 16)]
                for j in range(1, R):
                    v = v + acc_v[p * R + j, pl.ds(k * 16, 16)]
                obuf[c * P + p, pl.ds(k * 16, 16)] = v
            return 0

        lax.fori_loop(0, P, red, 0)
        return carry

    lax.fori_loop(0, NCH, chunk, 0)
    pltpu.sync_copy(obuf, out.at[pl.ds(wid * BPW, BPW)])


def _proj_body(p_ref, w_ref, b_ref, o_ref):
    acc = lax.dot_general(
        p_ref[...],
        w_ref[...],
        (((1,), (1,)), ((), ())),
        preferred_element_type=jnp.float32,
        precision=lax.Precision.HIGHEST,
    )
    o_ref[...] = acc * (1.0 / S) + b_ref[...]


def kernel(x, emb_table, W, b):
    x_lin = x.reshape(NW * XROWS, XW)

    mesh = plsc.VectorSubcoreMesh(
        core_axis_name="c", subcore_axis_name="s", num_cores=NC, num_subcores=NS
    )
    pooled_sums = pl.kernel(
        _pool_body,
        out_type=jax.ShapeDtypeStruct((B, D), jnp.float32),
        mesh=mesh,
        compiler_params=pltpu.CompilerParams(
            use_tc_tiling_on_sc=False, needs_layout_passes=False
        ),
        scratch_types=[
            pltpu.VMEM((XROWS, XW), jnp.int32),
            pltpu.VMEM((G * GSZ,), jnp.int32),
            pltpu.VMEM((GSZ, D), jnp.float32),
            pltpu.VMEM((BPW, D), jnp.float32),
            pltpu.SemaphoreType.DMA,
        ],
    )(x_lin, emb_table)

    out = pl.pallas_call(
        _proj_body,
        out_shape=jax.ShapeDtypeStruct((B, C_OUT), jnp.float32),
    )(pooled_sums, W, b.reshape(1, C_OUT))
    return out


# speedup vs baseline: 1.0488x; 1.0123x over previous
import jax
import jax.numpy as jnp
from jax import lax
from jax.experimental import pallas as pl
from jax.experimental.pallas import tpu as pltpu
from jax.experimental.pallas import tpu_sc as plsc

B = 4096
S = 200
D = 64
C_OUT = 2
NC, NS = 2, 16
NW = NC * NS
BPW = B // NW
P = 16
R = 8
G = S // R
GSZ = P * R
NCH = BPW // P
LG = D // 16
XW = 128
XROWS = BPW * S // XW


def _pool_body(xl_hbm, table, out, xrows, idx2a, idx2b, acca, accb, obuf,
               sema, semb):
    wid = lax.axis_index("s") * NC + lax.axis_index("c")
    pltpu.sync_copy(xl_hbm.at[pl.ds(wid * XROWS, XROWS)], xrows)
    zero = jnp.zeros((16,), jnp.float32)
    lane = jax.lax.iota(jnp.int32, 16)
    lane_hi = lane >> 3
    lane_lo = lane & 7

    idx2s = (idx2a, idx2b)
    accs = (acca, accb)
    sems = (sema, semb)

    def build(c, idx2):
        def bld_g(g, _):
            def bld_p(p2, _):
                flat = (c * P + p2 * 2 + lane_hi) * S + g * R + lane_lo
                v = plsc.load_gather(xrows, [flat >> 7, flat & 127])
                idx2[pl.ds(g * GSZ + p2 * 16, 16)] = v
                return 0

            lax.fori_loop(0, P // 2, bld_p, 0)
            return 0

        lax.fori_loop(0, G, bld_g, 0)

    def zero_fire(idx2, acc_v, sem):
        def zbody(r, _):
            for k in range(LG):
                acc_v[r, pl.ds(k * 16, 16)] = zero
            return 0

        lax.fori_loop(0, GSZ, zbody, 0)

        def fire(g, _):
            pltpu.async_copy(
                table.at[idx2.at[pl.ds(g * GSZ, GSZ)]],
                acc_v,
                sem,
                add=True,
            )
            return 0

        lax.fori_loop(0, G, fire, 0)

    def drain_reduce(c, idx2, acc_v, sem):
        def drain(g, _):
            pltpu.make_async_copy(
                table.at[idx2.at[pl.ds(0, GSZ)]], acc_v, sem
            ).wait()
            return 0

        lax.fori_loop(0, G, drain, 0)

        def red(p, _):
            for k in range(LG):
                v = acc_v[p * R, pl.ds(k * 16, 16)]
                for j in range(1, R):
                    v = v + acc_v[p * R + j, pl.ds(k * 16, 16)]
                obuf[c * P + p, pl.ds(k * 16, 16)] = v
            return 0

        lax.fori_loop(0, P, red, 0)

    build(0, idx2s[0])
    zero_fire(idx2s[0], accs[0], sems[0])
    for c in range(1, NCH):
        b = c % 2
        build(c, idx2s[b])
        zero_fire(idx2s[b], accs[b], sems[b])
        drain_reduce(c - 1, idx2s[1 - b], accs[1 - b], sems[1 - b])
    last = (NCH - 1) % 2
    drain_reduce(NCH - 1, idx2s[last], accs[last], sems[last])

    pltpu.sync_copy(obuf, out.at[pl.ds(wid * BPW, BPW)])


def _proj_body(p_ref, w_ref, b_ref, o_ref):
    acc = lax.dot_general(
        p_ref[...],
        w_ref[...],
        (((1,), (1,)), ((), ())),
        preferred_element_type=jnp.float32,
        precision=lax.Precision.HIGHEST,
    )
    o_ref[...] = acc * (1.0 / S) + b_ref[...]


def kernel(x, emb_table, W, b):
    x_lin = x.reshape(NW * XROWS, XW)

    mesh = plsc.VectorSubcoreMesh(
        core_axis_name="c", subcore_axis_name="s", num_cores=NC, num_subcores=NS
    )
    pooled_sums = pl.kernel(
        _pool_body,
        out_type=jax.ShapeDtypeStruct((B, D), jnp.float32),
        mesh=mesh,
        compiler_params=pltpu.CompilerParams(
            use_tc_tiling_on_sc=False, needs_layout_passes=False
        ),
        scratch_types=[
            pltpu.VMEM((XROWS, XW), jnp.int32),
            pltpu.VMEM((G * GSZ,), jnp.int32),
            pltpu.VMEM((G * GSZ,), jnp.int32),
            pltpu.VMEM((GSZ, D), jnp.float32),
            pltpu.VMEM((GSZ, D), jnp.float32),
            pltpu.VMEM((BPW, D), jnp.float32),
            pltpu.SemaphoreType.DMA,
            pltpu.SemaphoreType.DMA,
        ],
    )(x_lin, emb_table)

    out = pl.pallas_call(
        _proj_body,
        out_shape=jax.ShapeDtypeStruct((B, C_OUT), jnp.float32),
    )(pooled_sums, W, b.reshape(1, C_OUT))
    return out
